# baseline (speedup 1.0000x reference)
import functools

import jax
import jax.numpy as jnp
from jax import lax
from jax.experimental import pallas as pl
from jax.experimental.pallas import tpu as pltpu
from jax.experimental.pallas import tpu_sc as plsc

B = 4096
S = 200
D = 64
C = 2
HALF = S // 2
NC = 2
NS = 16
NW = NC * NS
BPW = B // NW
INV_S = 1.0 / S

def _body(ids_hbm, table_hbm, w_hbm, b_hbm, out_hbm,
          ids_v, rows_a, rows_b, w_v, b_v, out_v, sem_a, sem_b):
    wid = lax.axis_index("s") * NC + lax.axis_index("c")
    base = wid * BPW

    pltpu.sync_copy(ids_hbm.at[pl.ds(base * 2, BPW * 2)], ids_v)
    pltpu.sync_copy(w_hbm, w_v)
    pltpu.sync_copy(b_hbm, b_v)

    bufs = ((rows_a, sem_a), (rows_b, sem_b))

    def issue(e, buf, sem):
        pltpu.async_copy(table_hbm.at[ids_v.at[2 * e]],
                         buf.at[pl.ds(0, HALF)], sem)
        pltpu.async_copy(table_hbm.at[ids_v.at[2 * e + 1]],
                         buf.at[pl.ds(HALF, HALF)], sem)

    def drain(e, buf, sem):
        pltpu.make_async_copy(table_hbm.at[ids_v.at[2 * e]],
                              buf.at[pl.ds(0, HALF)], sem).wait()
        pltpu.make_async_copy(table_hbm.at[ids_v.at[2 * e + 1]],
                              buf.at[pl.ds(HALF, HALF)], sem).wait()

    for k, (buf, sem) in enumerate(bufs):
        issue(k, buf, sem)

    @pl.loop(0, BPW, step=2)
    def _elements(e0):
        for k, (buf, sem) in enumerate(bufs):
            e = e0 + k
            drain(e, buf, sem)

            zero = jnp.zeros((16,), jnp.float32)

            @plsc.parallel_loop(0, S, carry=(zero, zero, zero, zero),
                                unroll=4)
            def acc(r, carry):
                a0, a1, a2, a3 = carry
                return (a0 + buf[r, pl.ds(0, 16)],
                        a1 + buf[r, pl.ds(16, 16)],
                        a2 + buf[r, pl.ds(32, 16)],
                        a3 + buf[r, pl.ds(48, 16)])

            a0, a1, a2, a3 = acc

            @pl.when(e + 2 < BPW)
            def _():
                issue(e + 2, buf, sem)

            tot = []
            for c in range(C):
                s = (a0 * w_v[c, pl.ds(0, 16)] +
                     a1 * w_v[c, pl.ds(16, 16)] +
                     a2 * w_v[c, pl.ds(32, 16)] +
                     a3 * w_v[c, pl.ds(48, 16)])
                tot.append(jnp.sum(s) * INV_S)

            lane = lax.iota(jnp.int32, 16)
            res = jnp.where(lane == 0, tot[0],
                            jnp.where(lane == 1, tot[1], 0.0))
            res = res + b_v[...]
            out_v[pl.ds(C * e, 16)] = res

    pltpu.sync_copy(out_v.at[pl.ds(0, BPW * C)],
                    out_hbm.at[pl.ds(base * C, BPW * C)])


@functools.cache
def _build():
    mesh = plsc.VectorSubcoreMesh(core_axis_name="c", subcore_axis_name="s",
                                  num_cores=NC, num_subcores=NS)

    @functools.partial(
        pl.kernel,
        out_type=jax.ShapeDtypeStruct((B * C,), jnp.float32),
        mesh=mesh,
        compiler_params=pltpu.CompilerParams(needs_layout_passes=False,
                                             use_tc_tiling_on_sc=False),
        scratch_types=[
            pltpu.VMEM((BPW * 2, HALF), jnp.int32),
            pltpu.VMEM((S, D), jnp.float32),
            pltpu.VMEM((S, D), jnp.float32),
            pltpu.VMEM((C, D), jnp.float32),
            pltpu.VMEM((16,), jnp.float32),
            pltpu.VMEM((BPW * C + 16,), jnp.float32),
            pltpu.SemaphoreType.DMA,
            pltpu.SemaphoreType.DMA,
        ],
    )
    def _embedbag_head(ids_hbm, table_hbm, w_hbm, b_hbm, out_hbm, *scratch):
        _body(ids_hbm, table_hbm, w_hbm, b_hbm, out_hbm, *scratch)

    return _embedbag_head


def kernel(input_ids, emb_table, head_W, head_b):
    ids = input_ids.astype(jnp.int32).reshape(B * 2, HALF)
    b16 = jnp.zeros((16,), jnp.float32).at[:C].set(head_b)
    out = _build()(ids, emb_table, head_W, b16)
    return out.reshape(B, C)


# speedup vs baseline: 1.8970x; 1.8970x over previous
import functools

import jax
import jax.numpy as jnp
from jax import lax
from jax.experimental import pallas as pl
from jax.experimental.pallas import tpu as pltpu
from jax.experimental.pallas import tpu_sc as plsc

B = 4096
S = 200
D = 64
C = 2
V = 1000000
HALF = S // 2
HPAD = 104
GBUF = 2 * HPAD
NC = 2
NS = 16
NW = NC * NS
BPW = B // NW
INV_S = 1.0 / S
NB = 8192


def _proj_body(w_ref, t_ref, p0_ref, p1_ref):
    m = lax.dot_general(w_ref[...], t_ref[...], (((1,), (0,)), ((), ())),
                        precision=lax.Precision.HIGHEST,
                        preferred_element_type=jnp.float32)
    p0_ref[...] = m[0]
    p1_ref[...] = m[1]


def _project(head_W, table_t):
    return pl.pallas_call(
        _proj_body,
        grid=(pl.cdiv(V, NB),),
        in_specs=[
            pl.BlockSpec((C, D), lambda k: (0, 0)),
            pl.BlockSpec((D, NB), lambda k: (0, k)),
        ],
        out_specs=[
            pl.BlockSpec((NB,), lambda k: (k,)),
            pl.BlockSpec((NB,), lambda k: (k,)),
        ],
        out_shape=[
            jax.ShapeDtypeStruct((V,), jnp.float32),
            jax.ShapeDtypeStruct((V,), jnp.float32),
        ],
    )(head_W, table_t)


def _sc_body(ids_hbm, p0_hbm, p1_hbm, b_hbm, out_hbm,
             ids_v, d0, d1, b_v, out_v, sem00, sem01, sem10, sem11):
    wid = lax.axis_index("s") * NC + lax.axis_index("c")
    base = wid * BPW

    pltpu.sync_copy(ids_hbm.at[pl.ds(base * 2, BPW * 2)], ids_v)
    pltpu.sync_copy(b_hbm, b_v)

    sems = ((sem00, sem01), (sem10, sem11))

    def issue(e, k):
        for p_hbm, dbuf, sem in ((p0_hbm, d0, sems[0][k]),
                                 (p1_hbm, d1, sems[1][k])):
            pltpu.async_copy(p_hbm.at[ids_v.at[2 * e]],
                             dbuf.at[k, pl.ds(0, HPAD)], sem)
            pltpu.async_copy(p_hbm.at[ids_v.at[2 * e + 1]],
                             dbuf.at[k, pl.ds(HPAD, HPAD)], sem)

    def drain(e, k):
        for p_hbm, dbuf, sem in ((p0_hbm, d0, sems[0][k]),
                                 (p1_hbm, d1, sems[1][k])):
            pltpu.make_async_copy(p_hbm.at[ids_v.at[2 * e]],
                                  dbuf.at[k, pl.ds(0, HPAD)], sem).wait()
            pltpu.make_async_copy(p_hbm.at[ids_v.at[2 * e + 1]],
                                  dbuf.at[k, pl.ds(HPAD, HPAD)], sem).wait()

    issue(0, 0)
    issue(1, 1)

    lane = lax.iota(jnp.int32, 16)

    @pl.loop(0, BPW, step=2)
    def _elements(e0):
        for k in range(2):
            e = e0 + k
            drain(e, k)

            tot = []
            for dbuf in (d0, d1):
                acc = dbuf[k, pl.ds(0, 16)]
                for i in range(1, GBUF // 16):
                    acc = acc + dbuf[k, pl.ds(16 * i, 16)]
                tot.append(jnp.sum(acc) * INV_S)

            @pl.when(e + 2 < BPW)
            def _():
                issue(e + 2, k)

            res = jnp.where(lane == 0, tot[0],
                            jnp.where(lane == 1, tot[1], 0.0))
            res = res + b_v[...]
            out_v[pl.ds(C * e, 16)] = res

    pltpu.sync_copy(out_v.at[pl.ds(0, BPW * C)],
                    out_hbm.at[pl.ds(base * C, BPW * C)])


@functools.cache
def _build_sc():
    mesh = plsc.VectorSubcoreMesh(core_axis_name="c", subcore_axis_name="s",
                                  num_cores=NC, num_subcores=NS)

    @functools.partial(
        pl.kernel,
        out_type=jax.ShapeDtypeStruct((B * C,), jnp.float32),
        mesh=mesh,
        compiler_params=pltpu.CompilerParams(needs_layout_passes=False,
                                             use_tc_tiling_on_sc=False),
        scratch_types=[
            pltpu.VMEM((BPW * 2, HPAD), jnp.int32),
            pltpu.VMEM((2, GBUF), jnp.float32),
            pltpu.VMEM((2, GBUF), jnp.float32),
            pltpu.VMEM((16,), jnp.float32),
            pltpu.VMEM((BPW * C + 16,), jnp.float32),
            pltpu.SemaphoreType.DMA,
            pltpu.SemaphoreType.DMA,
            pltpu.SemaphoreType.DMA,
            pltpu.SemaphoreType.DMA,
        ],
    )
    def _gather_pool(ids_hbm, p0_hbm, p1_hbm, b_hbm, out_hbm, *scratch):
        _sc_body(ids_hbm, p0_hbm, p1_hbm, b_hbm, out_hbm, *scratch)

    return _gather_pool


def kernel(input_ids, emb_table, head_W, head_b):
    p0, p1 = _project(head_W, emb_table.T)
    p0 = jnp.pad(p0, (0, 8))
    p1 = jnp.pad(p1, (0, 8))
    ids = input_ids.astype(jnp.int32).reshape(B * 2, HALF)
    ids = jnp.pad(ids, ((0, 0), (0, HPAD - HALF)), constant_values=V)
    b16 = jnp.zeros((16,), jnp.float32).at[:C].set(head_b)
    out = _build_sc()(ids, p0, p1, b16)
    return out.reshape(B, C)


# speedup vs baseline: 2.3419x; 1.2345x over previous
import functools

import jax
import jax.numpy as jnp
from jax import lax
from jax.experimental import pallas as pl
from jax.experimental.pallas import tpu as pltpu
from jax.experimental.pallas import tpu_sc as plsc

B = 4096
S = 200
D = 64
C = 2
V = 1000000
HALF = S // 2
HPAD = 104
GBUF = 2 * HPAD
NC = 2
NS = 16
NW = NC * NS
BPW = B // NW
INV_S = 1.0 / S
NB = 16384
NDEEP = 4


def _proj_body(w_ref, t_ref, p0_ref, p1_ref):
    m = lax.dot_general(w_ref[...], t_ref[...], (((1,), (0,)), ((), ())),
                        preferred_element_type=jnp.float32)
    p0_ref[...] = m[0]
    p1_ref[...] = m[1]


def _project(head_W, table_t):
    return pl.pallas_call(
        _proj_body,
        grid=(pl.cdiv(V, NB),),
        in_specs=[
            pl.BlockSpec((C, D), lambda k: (0, 0)),
            pl.BlockSpec((D, NB), lambda k: (0, k)),
        ],
        out_specs=[
            pl.BlockSpec((NB,), lambda k: (k,)),
            pl.BlockSpec((NB,), lambda k: (k,)),
        ],
        out_shape=[
            jax.ShapeDtypeStruct((V,), jnp.float32),
            jax.ShapeDtypeStruct((V,), jnp.float32),
        ],
    )(head_W, table_t)


def _sc_body(ids_hbm, p0_hbm, p1_hbm, b_hbm, out_hbm,
             ids_v, d0, d1, b_v, out_v, *sems):
    wid = lax.axis_index("s") * NC + lax.axis_index("c")
    base = wid * BPW

    pltpu.sync_copy(ids_hbm.at[pl.ds(base * 2, BPW * 2)], ids_v)
    pltpu.sync_copy(b_hbm, b_v)

    def issue(e, k):
        for c, (p_hbm, dbuf) in enumerate(((p0_hbm, d0), (p1_hbm, d1))):
            sem = sems[c * NDEEP + k]
            pltpu.async_copy(p_hbm.at[ids_v.at[2 * e]],
                             dbuf.at[k, pl.ds(0, HPAD)], sem)
            pltpu.async_copy(p_hbm.at[ids_v.at[2 * e + 1]],
                             dbuf.at[k, pl.ds(HPAD, HPAD)], sem)

    def drain(e, k):
        for c, (p_hbm, dbuf) in enumerate(((p0_hbm, d0), (p1_hbm, d1))):
            sem = sems[c * NDEEP + k]
            pltpu.make_async_copy(p_hbm.at[ids_v.at[2 * e]],
                                  dbuf.at[k, pl.ds(0, HPAD)], sem).wait()
            pltpu.make_async_copy(p_hbm.at[ids_v.at[2 * e + 1]],
                                  dbuf.at[k, pl.ds(HPAD, HPAD)], sem).wait()

    for k in range(NDEEP):
        issue(k, k)

    lane = lax.iota(jnp.int32, 16)

    @pl.loop(0, BPW, step=NDEEP)
    def _elements(e0):
        for k in range(NDEEP):
            e = e0 + k
            drain(e, k)

            tot = []
            for dbuf in (d0, d1):
                acc = dbuf[k, pl.ds(0, 16)]
                for i in range(1, GBUF // 16):
                    acc = acc + dbuf[k, pl.ds(16 * i, 16)]
                tot.append(jnp.sum(acc) * INV_S)

            @pl.when(e + NDEEP < BPW)
            def _():
                issue(e + NDEEP, k)

            res = jnp.where(lane == 0, tot[0],
                            jnp.where(lane == 1, tot[1], 0.0))
            res = res + b_v[...]
            out_v[pl.ds(C * e, 16)] = res

    pltpu.sync_copy(out_v.at[pl.ds(0, BPW * C)],
                    out_hbm.at[pl.ds(base * C, BPW * C)])


@functools.cache
def _build_sc():
    mesh = plsc.VectorSubcoreMesh(core_axis_name="c", subcore_axis_name="s",
                                  num_cores=NC, num_subcores=NS)

    @functools.partial(
        pl.kernel,
        out_type=jax.ShapeDtypeStruct((B * C,), jnp.float32),
        mesh=mesh,
        compiler_params=pltpu.CompilerParams(needs_layout_passes=False,
                                             use_tc_tiling_on_sc=False),
        scratch_types=[
            pltpu.VMEM((BPW * 2, HPAD), jnp.int32),
            pltpu.VMEM((NDEEP, GBUF), jnp.float32),
            pltpu.VMEM((NDEEP, GBUF), jnp.float32),
            pltpu.VMEM((16,), jnp.float32),
            pltpu.VMEM((BPW * C + 16,), jnp.float32),
        ] + [pltpu.SemaphoreType.DMA] * (2 * NDEEP),
    )
    def _gather_pool(ids_hbm, p0_hbm, p1_hbm, b_hbm, out_hbm, *scratch):
        _sc_body(ids_hbm, p0_hbm, p1_hbm, b_hbm, out_hbm, *scratch)

    return _gather_pool


def kernel(input_ids, emb_table, head_W, head_b):
    p0, p1 = _project(head_W, emb_table.T)
    p0 = jnp.pad(p0, (0, 8))
    p1 = jnp.pad(p1, (0, 8))
    ids = input_ids.astype(jnp.int32).reshape(B * 2, HALF)
    ids = jnp.pad(ids, ((0, 0), (0, HPAD - HALF)), constant_values=V)
    b16 = jnp.zeros((16,), jnp.float32).at[:C].set(head_b)
    out = _build_sc()(ids, p0, p1, b16)
    return out.reshape(B, C)


# speedup vs baseline: 2.4711x; 1.0552x over previous
import functools

import jax
import jax.numpy as jnp
from jax import lax
from jax.experimental import pallas as pl
from jax.experimental.pallas import tpu as pltpu
from jax.experimental.pallas import tpu_sc as plsc

B = 4096
S = 200
D = 64
C = 2
V = 1000000
HALF = S // 2
HPAD = 104
GBUF = 2 * HPAD
NC = 2
NS = 16
NW = NC * NS
BPW = B // NW
INV_S = 1.0 / S
NB = 16384
NDEEP = 4


def _proj_body(w_ref, t_ref, p_ref):
    m = lax.dot_general(w_ref[...], t_ref[...], (((1,), (0,)), ((), ())),
                        preferred_element_type=jnp.float32)
    r0 = m[0].astype(jnp.bfloat16).astype(jnp.float32)
    r1 = m[1].astype(jnp.bfloat16).astype(jnp.float32)
    b0 = lax.shift_right_logical(lax.bitcast_convert_type(r0, jnp.int32), 16)
    b1 = jnp.bitwise_and(lax.bitcast_convert_type(r1, jnp.int32),
                         jnp.int32(-65536))
    p_ref[...] = jnp.bitwise_or(b0, b1)


def _project(head_W, table_t):
    return pl.pallas_call(
        _proj_body,
        grid=(pl.cdiv(V, NB),),
        in_specs=[
            pl.BlockSpec((C, D), lambda k: (0, 0)),
            pl.BlockSpec((D, NB), lambda k: (0, k)),
        ],
        out_specs=[
            pl.BlockSpec((NB,), lambda k: (k,)),
        ],
        out_shape=[
            jax.ShapeDtypeStruct((V,), jnp.int32),
        ],
    )(head_W, table_t)


def _sc_body(ids_hbm, p_hbm, b_hbm, out_hbm,
             ids_v, dbuf, b_v, out_v, *sems):
    wid = lax.axis_index("s") * NC + lax.axis_index("c")
    base = wid * BPW

    pltpu.sync_copy(ids_hbm.at[pl.ds(base * 2, BPW * 2)], ids_v)
    pltpu.sync_copy(b_hbm, b_v)

    def issue(e, k):
        pltpu.async_copy(p_hbm.at[ids_v.at[2 * e]],
                         dbuf.at[k, pl.ds(0, HPAD)], sems[k])
        pltpu.async_copy(p_hbm.at[ids_v.at[2 * e + 1]],
                         dbuf.at[k, pl.ds(HPAD, HPAD)], sems[k])

    def drain(e, k):
        pltpu.make_async_copy(p_hbm.at[ids_v.at[2 * e]],
                              dbuf.at[k, pl.ds(0, HPAD)], sems[k]).wait()
        pltpu.make_async_copy(p_hbm.at[ids_v.at[2 * e + 1]],
                              dbuf.at[k, pl.ds(HPAD, HPAD)], sems[k]).wait()

    for k in range(NDEEP):
        issue(k, k)

    lane = lax.iota(jnp.int32, 16)
    himask = jnp.full((16,), -65536, jnp.int32)

    @pl.loop(0, BPW, step=NDEEP)
    def _elements(e0):
        for k in range(NDEEP):
            e = e0 + k
            drain(e, k)

            acc0 = jnp.zeros((16,), jnp.float32)
            acc1 = jnp.zeros((16,), jnp.float32)
            for i in range(GBUF // 16):
                w = dbuf[k, pl.ds(16 * i, 16)]
                acc0 = acc0 + lax.bitcast_convert_type(
                    lax.shift_left(w, 16), jnp.float32)
                acc1 = acc1 + lax.bitcast_convert_type(
                    jnp.bitwise_and(w, himask), jnp.float32)
            tot0 = jnp.sum(acc0) * INV_S
            tot1 = jnp.sum(acc1) * INV_S

            @pl.when(e + NDEEP < BPW)
            def _():
                issue(e + NDEEP, k)

            res = jnp.where(lane == 0, tot0,
                            jnp.where(lane == 1, tot1, 0.0))
            res = res + b_v[...]
            out_v[pl.ds(C * e, 16)] = res

    pltpu.sync_copy(out_v.at[pl.ds(0, BPW * C)],
                    out_hbm.at[pl.ds(base * C, BPW * C)])


@functools.cache
def _build_sc():
    mesh = plsc.VectorSubcoreMesh(core_axis_name="c", subcore_axis_name="s",
                                  num_cores=NC, num_subcores=NS)

    @functools.partial(
        pl.kernel,
        out_type=jax.ShapeDtypeStruct((B * C,), jnp.float32),
        mesh=mesh,
        compiler_params=pltpu.CompilerParams(needs_layout_passes=False,
                                             use_tc_tiling_on_sc=False),
        scratch_types=[
            pltpu.VMEM((BPW * 2, HPAD), jnp.int32),
            pltpu.VMEM((NDEEP, GBUF), jnp.int32),
            pltpu.VMEM((16,), jnp.float32),
            pltpu.VMEM((BPW * C + 16,), jnp.float32),
        ] + [pltpu.SemaphoreType.DMA] * NDEEP,
    )
    def _gather_pool(ids_hbm, p_hbm, b_hbm, out_hbm, *scratch):
        _sc_body(ids_hbm, p_hbm, b_hbm, out_hbm, *scratch)

    return _gather_pool


def kernel(input_ids, emb_table, head_W, head_b):
    (p,) = _project(head_W, emb_table.T)
    p = jnp.pad(p, (0, 8))
    ids = input_ids.astype(jnp.int32).reshape(B * 2, HALF)
    ids = jnp.pad(ids, ((0, 0), (0, HPAD - HALF)), constant_values=V)
    b16 = jnp.zeros((16,), jnp.float32).at[:C].set(head_b)
    out = _build_sc()(ids, p, b16)
    return out.reshape(B, C)
